# Initial kernel scaffold; baseline (speedup 1.0000x reference)
import jax
import jax.numpy as jnp
from jax.experimental import pallas as pl


def kernel(x, params):
    raise NotImplementedError("write your pallas kernel here")


# speedup vs baseline: 7.7918x; 7.7918x over previous
import functools

import jax
import jax.numpy as jnp
from jax import lax
from jax.experimental import pallas as pl
from jax.experimental.pallas import tpu as pltpu
from jax.experimental.pallas import tpu_sc as plsc

KNB = 20
EPS = 1e-5
SLOPE = 0.2


def _lrelu(x):
    return jnp.where(x >= 0, x, SLOPE * x)


def _knn_body(q_ref, t_ref, idx_ref, *, R, N, nk):
    b = pl.program_id(0)
    q = q_ref[0]
    t = t_ref[0]
    d = lax.dot_general(q, t, (((1,), (1,)), ((), ())),
                        preferred_element_type=jnp.float32)
    qn = jnp.sum(q * q, axis=1, keepdims=True)
    tn = jnp.sum(t * t, axis=1)[None, :]
    pd = 2.0 * d - qn - tn
    iota = lax.broadcasted_iota(jnp.int32, (R, N), 1)
    base = b * N
    for k in range(nk):
        m = jnp.max(pd, axis=1, keepdims=True)
        am = jnp.min(jnp.where(pd >= m, iota, N), axis=1)
        idx_ref[0, :, k] = am + base
        pd = jnp.where(iota == am[:, None], -jnp.inf, pd)


def _knn(xt, R=256):
    B, N, C = xt.shape
    return pl.pallas_call(
        functools.partial(_knn_body, R=R, N=N, nk=KNB),
        grid=(B, N // R),
        in_specs=[
            pl.BlockSpec((1, R, C), lambda b, r: (b, r, 0)),
            pl.BlockSpec((1, N, C), lambda b, r: (b, 0, 0)),
        ],
        out_specs=pl.BlockSpec((1, R, KNB), lambda b, r: (b, r, 0)),
        out_shape=jax.ShapeDtypeStruct((B, N, KNB), jnp.int32),
    )(xt, xt)


def _sc_gather(table, idx_flat):
    M = idx_flat.shape[0]
    D = table.shape[1]
    info = plsc.get_sparse_core_info()
    NW = info.num_cores * info.num_subcores
    CH = 128
    m_per = M // NW
    iters = m_per // CH
    mesh = plsc.VectorSubcoreMesh(core_axis_name="c", subcore_axis_name="s")

    @functools.partial(
        pl.kernel,
        mesh=mesh,
        compiler_params=pltpu.CompilerParams(use_tc_tiling_on_sc=False),
        out_type=jax.ShapeDtypeStruct((M, D), jnp.float32),
        scratch_types=[
            pltpu.VMEM((CH,), jnp.int32),
            pltpu.VMEM((CH, D), jnp.float32),
            pltpu.SemaphoreType.DMA,
        ],
    )
    def k(table_hbm, idx_hbm, out_hbm, idx_v, rows_v, sem):
        wid = lax.axis_index("s") * info.num_cores + lax.axis_index("c")
        base0 = wid * m_per

        def body(i, carry):
            base = base0 + i * CH
            pltpu.sync_copy(idx_hbm.at[pl.ds(base, CH)], idx_v)
            pltpu.async_copy(table_hbm.at[idx_v], rows_v, sem).wait()
            pltpu.sync_copy(rows_v, out_hbm.at[pl.ds(base, CH)])
            return carry

        lax.fori_loop(0, iters, body, 0)

    return k(table, idx_flat)


def _bact(y, st_ref, gb_ref):
    return _lrelu((y - st_ref[0, :]) / st_ref[1, :] * gb_ref[0, :]
                  + gb_ref[1, :])


def _acc_stats(stats_ref, part_ref, m0_ref, y, p, P, M):
    axes = tuple(range(y.ndim - 1))
    cnt = 1.0
    for d_ in y.shape[:-1]:
        cnt *= d_

    @pl.when(p == 0)
    def _():
        m0_ref[0, :] = jnp.sum(y, axis=axes) / cnt

    m0 = m0_ref[0, :]
    yc = y - m0
    part_ref[0, p, :] = jnp.sum(yc, axis=axes)
    part_ref[1, p, :] = jnp.sum(yc * yc, axis=axes)

    @pl.when(p == P - 1)
    def _():
        s = jnp.sum(part_ref[0], axis=0)
        sq = jnp.sum(part_ref[1], axis=0)
        dm = s / M
        var = jnp.maximum(sq / M - dm * dm, 0.0)
        stats_ref[0, :] = m0 + dm
        stats_ref[1, :] = jnp.sqrt(var + EPS)


def _edge_feat(g_ref, q_ref, C, R, nk):
    D = g_ref.shape[1]
    f = g_ref[...].reshape(R, nk, D)
    q = q_ref[...]
    fm = f - q[:, None, :]
    feat = jnp.concatenate(
        [fm[:, :, :C], jnp.broadcast_to(q[:, None, :C], (R, nk, C))], axis=2)
    return feat.reshape(R * nk, 2 * C)


def _edge_stats1_body(g_ref, q_ref, w1_ref, stats_ref, part_ref, m0_ref, *,
                      C, R, nk, M, P):
    p = pl.program_id(0)
    y1 = lax.dot_general(_edge_feat(g_ref, q_ref, C, R, nk), w1_ref[...],
                         (((1,), (1,)), ((), ())),
                         preferred_element_type=jnp.float32)
    _acc_stats(stats_ref, part_ref, m0_ref, y1, p, P, M)


def _edge_stats2_body(g_ref, q_ref, w1_ref, st1_ref, gb1_ref, w2_ref,
                      stats_ref, part_ref, m0_ref, *, C, R, nk, M, P):
    p = pl.program_id(0)
    y1 = lax.dot_general(_edge_feat(g_ref, q_ref, C, R, nk), w1_ref[...],
                         (((1,), (1,)), ((), ())),
                         preferred_element_type=jnp.float32)
    z1 = _bact(y1, st1_ref, gb1_ref)
    y2 = lax.dot_general(z1, w2_ref[...], (((1,), (1,)), ((), ())),
                         preferred_element_type=jnp.float32)
    _acc_stats(stats_ref, part_ref, m0_ref, y2, p, P, M)


def _edge_final2_body(g_ref, q_ref, w1_ref, st1_ref, gb1_ref, st2_ref,
                      gb2_ref, w2_ref, out_ref, *, C, R, nk):
    y1 = lax.dot_general(_edge_feat(g_ref, q_ref, C, R, nk), w1_ref[...],
                         (((1,), (1,)), ((), ())),
                         preferred_element_type=jnp.float32)
    z1 = _bact(y1, st1_ref, gb1_ref)
    y2 = lax.dot_general(z1, w2_ref[...], (((1,), (1,)), ((), ())),
                         preferred_element_type=jnp.float32)
    z2 = _bact(y2, st2_ref, gb2_ref).reshape(R, nk, -1)
    out_ref[...] = jnp.max(z2, axis=1)


def _edge_final1_body(g_ref, q_ref, w1_ref, st1_ref, gb1_ref, out_ref, *,
                      C, R, nk):
    y1 = lax.dot_general(_edge_feat(g_ref, q_ref, C, R, nk), w1_ref[...],
                         (((1,), (1,)), ((), ())),
                         preferred_element_type=jnp.float32)
    z1 = _bact(y1, st1_ref, gb1_ref).reshape(R, nk, -1)
    out_ref[...] = jnp.max(z1, axis=1)


def _edge_stage(G, q_flat, C, w1, gb1, w2, gb2, R=512):
    BN, D = q_flat.shape
    P = BN // R
    M = float(BN * KNB)
    grid = (P,)
    g_spec = pl.BlockSpec((R * KNB, D), lambda p: (p, 0))
    q_spec = pl.BlockSpec((R, D), lambda p: (p, 0))
    st_spec = pl.BlockSpec((2, 64), lambda p: (0, 0))
    w1_spec = pl.BlockSpec(w1.shape, lambda p: (0, 0))
    w2_spec = pl.BlockSpec(w2.shape, lambda p: (0, 0))
    st_shape = jax.ShapeDtypeStruct((2, 64), jnp.float32)
    scr = [pltpu.VMEM((2, P, 64), jnp.float32), pltpu.VMEM((1, 64), jnp.float32)]

    st1 = pl.pallas_call(
        functools.partial(_edge_stats1_body, C=C, R=R, nk=KNB, M=M, P=P),
        grid=grid,
        in_specs=[g_spec, q_spec, w1_spec],
        out_specs=st_spec,
        out_shape=st_shape,
        scratch_shapes=scr,
    )(G, q_flat, w1)

    st2 = pl.pallas_call(
        functools.partial(_edge_stats2_body, C=C, R=R, nk=KNB, M=M, P=P),
        grid=grid,
        in_specs=[g_spec, q_spec, w1_spec, st_spec, st_spec, w2_spec],
        out_specs=st_spec,
        out_shape=st_shape,
        scratch_shapes=scr,
    )(G, q_flat, w1, st1, gb1, w2)

    x = pl.pallas_call(
        functools.partial(_edge_final2_body, C=C, R=R, nk=KNB),
        grid=grid,
        in_specs=[g_spec, q_spec, w1_spec, st_spec, st_spec, st_spec,
                  st_spec, w2_spec],
        out_specs=pl.BlockSpec((R, 64), lambda p: (p, 0)),
        out_shape=jax.ShapeDtypeStruct((BN, 64), jnp.float32),
    )(G, q_flat, w1, st1, gb1, st2, gb2, w2)
    return x


def _edge_stage1conv(G, q_flat, C, w1, gb1, R=512):
    BN, D = q_flat.shape
    P = BN // R
    M = float(BN * KNB)
    grid = (P,)
    g_spec = pl.BlockSpec((R * KNB, D), lambda p: (p, 0))
    q_spec = pl.BlockSpec((R, D), lambda p: (p, 0))
    st_spec = pl.BlockSpec((2, 64), lambda p: (0, 0))
    w1_spec = pl.BlockSpec(w1.shape, lambda p: (0, 0))
    st_shape = jax.ShapeDtypeStruct((2, 64), jnp.float32)
    scr = [pltpu.VMEM((2, P, 64), jnp.float32), pltpu.VMEM((1, 64), jnp.float32)]

    st1 = pl.pallas_call(
        functools.partial(_edge_stats1_body, C=C, R=R, nk=KNB, M=M, P=P),
        grid=grid,
        in_specs=[g_spec, q_spec, w1_spec],
        out_specs=st_spec,
        out_shape=st_shape,
        scratch_shapes=scr,
    )(G, q_flat, w1)

    x = pl.pallas_call(
        functools.partial(_edge_final1_body, C=C, R=R, nk=KNB),
        grid=grid,
        in_specs=[g_spec, q_spec, w1_spec, st_spec, st_spec],
        out_specs=pl.BlockSpec((R, 64), lambda p: (p, 0)),
        out_shape=jax.ShapeDtypeStruct((BN, 64), jnp.float32),
    )(G, q_flat, w1, st1, gb1)
    return x


def _tail_stats6_body(x1_ref, x2_ref, x3_ref, w6_ref, stats_ref,
                      part_ref, m0_ref, *, M, P):
    p = pl.program_id(0)
    h = jnp.concatenate([x1_ref[...], x2_ref[...], x3_ref[...]], axis=1)
    y = lax.dot_general(h, w6_ref[...], (((1,), (1,)), ((), ())),
                        preferred_element_type=jnp.float32)
    _acc_stats(stats_ref, part_ref, m0_ref, y, p, P, M)


def _tail_max6_body(x1_ref, x2_ref, x3_ref, w6_ref, st_ref, gb_ref,
                    out_ref, *, PR):
    r = pl.program_id(1)
    h = jnp.concatenate([x1_ref[...], x2_ref[...], x3_ref[...]], axis=1)
    y = lax.dot_general(h, w6_ref[...], (((1,), (1,)), ((), ())),
                        preferred_element_type=jnp.float32)
    z = _bact(y, st_ref, gb_ref)

    @pl.when(r == 0)
    def _():
        out_ref[...] = jnp.full_like(out_ref, -jnp.inf)

    out_ref[0, 0, :] = jnp.maximum(out_ref[0, 0, :], jnp.max(z, axis=0))


def _tail_y7(x1_ref, x2_ref, x3_ref, g_ref, w7g_ref, w7x_ref):
    h = jnp.concatenate([x1_ref[...], x2_ref[...], x3_ref[...]], axis=1)
    t = lax.dot_general(g_ref[0], w7g_ref[...], (((1,), (1,)), ((), ())),
                        preferred_element_type=jnp.float32)
    return lax.dot_general(h, w7x_ref[...], (((1,), (1,)), ((), ())),
                           preferred_element_type=jnp.float32) + t


def _tail_stats7_body(x1_ref, x2_ref, x3_ref, g_ref, w7g_ref, w7x_ref,
                      stats_ref, part_ref, m0_ref, *, M, P):
    p = pl.program_id(0)
    y = _tail_y7(x1_ref, x2_ref, x3_ref, g_ref, w7g_ref, w7x_ref)
    _acc_stats(stats_ref, part_ref, m0_ref, y, p, P, M)


def _tail_stats8_body(x1_ref, x2_ref, x3_ref, g_ref, w7g_ref, w7x_ref,
                      st7_ref, gb7_ref, w8_ref, stats_ref, part_ref,
                      m0_ref, *, M, P):
    p = pl.program_id(0)
    y7 = _tail_y7(x1_ref, x2_ref, x3_ref, g_ref, w7g_ref, w7x_ref)
    z7 = _bact(y7, st7_ref, gb7_ref)
    y8 = lax.dot_general(z7, w8_ref[...], (((1,), (1,)), ((), ())),
                         preferred_element_type=jnp.float32)
    _acc_stats(stats_ref, part_ref, m0_ref, y8, p, P, M)


def _tail_out_body(x1_ref, x2_ref, x3_ref, g_ref, w7g_ref, w7x_ref,
                   st7_ref, gb7_ref, w8_ref, st8_ref, gb8_ref, w9_ref,
                   out_ref):
    y7 = _tail_y7(x1_ref, x2_ref, x3_ref, g_ref, w7g_ref, w7x_ref)
    z7 = _bact(y7, st7_ref, gb7_ref)
    y8 = lax.dot_general(z7, w8_ref[...], (((1,), (1,)), ((), ())),
                         preferred_element_type=jnp.float32)
    z8 = _bact(y8, st8_ref, gb8_ref)
    out_ref[...] = lax.dot_general(z8, w9_ref[...], (((1,), (1,)), ((), ())),
                                   preferred_element_type=jnp.float32)


def kernel(x, params):
    p = params
    B, C, N = x.shape
    BN = B * N
    xt = jnp.swapaxes(x, 1, 2)

    def gb(i):
        return jnp.stack([p['g' + i], p['b' + i]])

    xt_p = jnp.pad(xt, ((0, 0), (0, 0), (0, 16 - C))).reshape(BN, 16)
    idx = _knn(xt)
    G = _sc_gather(xt_p, idx.reshape(BN * KNB))
    x1 = _edge_stage(G, xt_p, C, p['W1'], gb('1'), p['W2'], gb('2'))

    idx = _knn(x1.reshape(B, N, 64))
    G = _sc_gather(x1, idx.reshape(BN * KNB))
    x2 = _edge_stage(G, x1, 64, p['W3'], gb('3'), p['W4'], gb('4'))

    idx = _knn(x2.reshape(B, N, 64))
    G = _sc_gather(x2, idx.reshape(BN * KNB))
    x3 = _edge_stage1conv(G, x2, 64, p['W5'], gb('5'))

    out = _tail(x1, x2, x3, p, B, N)
    return out.reshape(B, N, 9)


def _tail(x1, x2, x3, p, B, N):
    BN = B * N

    def gb(i):
        return jnp.stack([p['g' + i], p['b' + i]])

    RT = 1024
    P = BN // RT
    xs_spec = pl.BlockSpec((RT, 64), lambda q: (q, 0))
    Mt = float(BN)

    st6 = pl.pallas_call(
        functools.partial(_tail_stats6_body, M=Mt, P=P),
        grid=(P,),
        in_specs=[xs_spec, xs_spec, xs_spec,
                  pl.BlockSpec((1024, 192), lambda q: (0, 0))],
        out_specs=pl.BlockSpec((2, 1024), lambda q: (0, 0)),
        out_shape=jax.ShapeDtypeStruct((2, 1024), jnp.float32),
        scratch_shapes=[pltpu.VMEM((2, P, 1024), jnp.float32),
                        pltpu.VMEM((1, 1024), jnp.float32)],
    )(x1, x2, x3, p['W6'])

    PR = N // RT
    xs_spec_b = pl.BlockSpec((RT, 64), lambda b, r: (b * PR + r, 0))
    g = pl.pallas_call(
        functools.partial(_tail_max6_body, PR=PR),
        grid=(B, PR),
        in_specs=[xs_spec_b, xs_spec_b, xs_spec_b,
                  pl.BlockSpec((1024, 192), lambda b, r: (0, 0)),
                  pl.BlockSpec((2, 1024), lambda b, r: (0, 0)),
                  pl.BlockSpec((2, 1024), lambda b, r: (0, 0))],
        out_specs=pl.BlockSpec((1, 1, 1024), lambda b, r: (b, 0, 0)),
        out_shape=jax.ShapeDtypeStruct((B, 1, 1024), jnp.float32),
    )(x1, x2, x3, p['W6'], st6, gb('6'))

    w7g = p['W7'][:, :1024]
    w7x = p['W7'][:, 1024:]
    g_spec = pl.BlockSpec((1, 1, 1024), lambda q: (q // PR, 0, 0))
    w7g_spec = pl.BlockSpec((512, 1024), lambda q: (0, 0))
    w7x_spec = pl.BlockSpec((512, 192), lambda q: (0, 0))
    st7_spec = pl.BlockSpec((2, 512), lambda q: (0, 0))
    st8_spec = pl.BlockSpec((2, 256), lambda q: (0, 0))

    st7 = pl.pallas_call(
        functools.partial(_tail_stats7_body, M=Mt, P=P),
        grid=(P,),
        in_specs=[xs_spec, xs_spec, xs_spec, g_spec, w7g_spec, w7x_spec],
        out_specs=st7_spec,
        out_shape=jax.ShapeDtypeStruct((2, 512), jnp.float32),
        scratch_shapes=[pltpu.VMEM((2, P, 512), jnp.float32),
                        pltpu.VMEM((1, 512), jnp.float32)],
    )(x1, x2, x3, g, w7g, w7x)

    st8 = pl.pallas_call(
        functools.partial(_tail_stats8_body, M=Mt, P=P),
        grid=(P,),
        in_specs=[xs_spec, xs_spec, xs_spec, g_spec, w7g_spec, w7x_spec,
                  st7_spec, st7_spec,
                  pl.BlockSpec((256, 512), lambda q: (0, 0))],
        out_specs=st8_spec,
        out_shape=jax.ShapeDtypeStruct((2, 256), jnp.float32),
        scratch_shapes=[pltpu.VMEM((2, P, 256), jnp.float32),
                        pltpu.VMEM((1, 256), jnp.float32)],
    )(x1, x2, x3, g, w7g, w7x, st7, gb('7'), p['W8'])

    out = pl.pallas_call(
        _tail_out_body,
        grid=(P,),
        in_specs=[xs_spec, xs_spec, xs_spec, g_spec, w7g_spec, w7x_spec,
                  st7_spec, st7_spec,
                  pl.BlockSpec((256, 512), lambda q: (0, 0)),
                  st8_spec, st8_spec,
                  pl.BlockSpec((9, 256), lambda q: (0, 0))],
        out_specs=pl.BlockSpec((RT, 9), lambda q: (q, 0)),
        out_shape=jax.ShapeDtypeStruct((BN, 9), jnp.float32),
    )(x1, x2, x3, g, w7g, w7x, st7, gb('7'), p['W8'], st8, gb('8'), p['W9'])

    return out
